# Initial kernel scaffold; baseline (speedup 1.0000x reference)
import jax
import jax.numpy as jnp
from jax.experimental import pallas as pl


def kernel(token_inputs, numerical_inputs, text_emb_table):
    raise NotImplementedError("write your pallas kernel here")


# speedup vs baseline: 3.8747x; 3.8747x over previous
import functools

import jax
import jax.numpy as jnp
from jax import lax
from jax.experimental import pallas as pl
from jax.experimental.pallas import tpu as pltpu
from jax.experimental.pallas import tpu_sc as plsc

VOCAB = 100000
EMB = 32
TOK = 8
LANES = 16
GW = 128
K = 2
CHUNK = K * GW
PACK = 128 // EMB


@functools.lru_cache(maxsize=None)
def _build(n_idx):
    info = plsc.get_sparse_core_info()
    nw = info.num_cores * info.num_subcores
    n_chunks = n_idx // CHUNK
    per_w = n_chunks // nw
    assert n_idx % CHUNK == 0 and n_chunks % nw == 0

    mesh = plsc.VectorSubcoreMesh(core_axis_name="c", subcore_axis_name="s")

    @functools.partial(
        pl.kernel,
        mesh=mesh,
        out_type=jax.ShapeDtypeStruct((n_idx // PACK, GW), jnp.float32),
        scratch_types=[
            pltpu.VMEM((K, GW), jnp.int32),
            pltpu.VMEM((CHUNK, GW), jnp.float32),
            pltpu.VMEM((CHUNK // PACK, GW), jnp.float32),
            pltpu.SemaphoreType.DMA,
        ],
    )
    def k(idx_hbm, table_hbm, out_hbm, idx_v, pad_v, comp_v, sem):
        wid = lax.axis_index("s") * info.num_cores + lax.axis_index("c")

        def chunk_body(i, carry):
            c = wid * per_w + i
            pltpu.sync_copy(idx_hbm.at[c], idx_v)
            for r in range(K):
                for o in range(GW // LANES):
                    v = idx_v[r, pl.ds(o * LANES, LANES)]
                    idx_v[r, pl.ds(o * LANES, LANES)] = jnp.clip(
                        v + 1, 0, VOCAB - 1
                    )
            copies = [
                pltpu.async_copy(
                    table_hbm.at[idx_v.at[r]],
                    pad_v.at[pl.ds(r * GW, GW)],
                    sem,
                )
                for r in range(K)
            ]
            for cp in copies:
                cp.wait()
            for t in range(CHUNK):
                for h in range(EMB // LANES):
                    comp_v[
                        t // PACK,
                        pl.ds((t % PACK) * EMB + h * LANES, LANES),
                    ] = pad_v[t, pl.ds(h * LANES, LANES)]
            pltpu.sync_copy(
                comp_v, out_hbm.at[pl.ds(c * (CHUNK // PACK), CHUNK // PACK)]
            )
            return carry

        lax.fori_loop(0, per_w, chunk_body, 0)

    return k


def kernel(token_inputs, numerical_inputs, text_emb_table):
    B, T, F = token_inputs.shape
    b_total = B * T
    n_idx = b_total * F
    idx = token_inputs.astype(jnp.int32).reshape(n_idx // CHUNK, K, GW)
    table_p = jnp.pad(text_emb_table, ((0, 0), (0, GW - EMB)))
    packed = _build(n_idx)(idx, table_p)
    flat_text = packed.reshape(b_total, F * EMB)
    num2d = numerical_inputs.reshape(b_total, -1)
    out = jnp.concatenate([flat_text, num2d], axis=-1)
    return out.reshape(B, T, F * EMB + numerical_inputs.shape[-1])


# speedup vs baseline: 6.0053x; 1.5499x over previous
import functools

import jax
import jax.numpy as jnp
from jax import lax
from jax.experimental import pallas as pl
from jax.experimental.pallas import tpu as pltpu
from jax.experimental.pallas import tpu_sc as plsc

VOCAB = 100000
EMB = 32
TOK = 8
NUMF = 48
OUTW = TOK * EMB + NUMF
LANES = 16
GW = 100
NG = 4
T = 50
HALF = NG * GW // 2


@functools.lru_cache(maxsize=None)
def _build(n_batch):
    info = plsc.get_sparse_core_info()
    nw = info.num_cores * info.num_subcores
    per_w = n_batch // nw
    assert n_batch % nw == 0

    mesh = plsc.VectorSubcoreMesh(core_axis_name="c", subcore_axis_name="s")

    CLIP_OFFS = (0, 16, 32, 48, 64, 80)

    @functools.partial(
        pl.kernel,
        mesh=mesh,
        out_type=jax.ShapeDtypeStruct((n_batch, T, OUTW), jnp.float32),
        scratch_types=[
            pltpu.VMEM((2, NG, GW), jnp.int32),
            pltpu.VMEM((2, HALF, 128), jnp.float32),
            pltpu.VMEM((T, NUMF), jnp.float32),
            pltpu.VMEM((T, OUTW), jnp.float32),
            pltpu.SemaphoreType.DMA,
            pltpu.SemaphoreType.DMA,
            pltpu.SemaphoreType.DMA,
            pltpu.SemaphoreType.DMA,
        ],
    )
    def k(idx_hbm, num_hbm, table_hbm, out_hbm,
          idx_v, pad_v, num_v, stage_v,
          sem_idx, sem_num, sem_g, sem_w):
        wid = lax.axis_index("s") * info.num_cores + lax.axis_index("c")
        b0 = wid * per_w

        pltpu.async_copy(idx_hbm.at[b0], idx_v.at[0], sem_idx)
        pltpu.async_copy(num_hbm.at[b0], num_v, sem_num)

        def body(i, carry):
            b = b0 + i
            ib = lax.rem(i, 2)
            pltpu.make_async_copy(idx_hbm.at[b], idx_v.at[ib], sem_idx).wait()
            lane = lax.iota(jnp.int32, LANES)
            for r in range(NG):
                for o in CLIP_OFFS:
                    v = idx_v[ib, r, pl.ds(o, LANES)]
                    idx_v[ib, r, pl.ds(o, LANES)] = jnp.clip(
                        v + 1, 0, VOCAB - 1
                    )
                v = idx_v[ib, r, pl.ds(GW - LANES, LANES)]
                idx_v[ib, r, pl.ds(GW - LANES, LANES)] = jnp.where(
                    lane < (96 - (GW - LANES)),
                    v,
                    jnp.clip(v + 1, 0, VOCAB - 1),
                )
            gathers = [
                pltpu.async_copy(
                    table_hbm.at[idx_v.at[ib, r]],
                    pad_v.at[r // 2, pl.ds((r % 2) * GW, GW)],
                    sem_g,
                )
                for r in range(NG)
            ]
            @pl.when(i + 1 < per_w)
            def _():
                pltpu.async_copy(
                    idx_hbm.at[b + 1], idx_v.at[1 - ib], sem_idx
                )

            @pl.when(i > 0)
            def _():
                pltpu.make_async_copy(stage_v, out_hbm.at[b], sem_w).wait()

            pltpu.make_async_copy(num_hbm.at[b], num_v, sem_num).wait()
            for j in range(T):
                for h in range(NUMF // LANES):
                    stage_v[j, pl.ds(TOK * EMB + h * LANES, LANES)] = num_v[
                        j, pl.ds(h * LANES, LANES)
                    ]
            for half in range(2):
                gathers[2 * half].wait()
                gathers[2 * half + 1].wait()
                for t in range(HALF):
                    n = half * HALF + t
                    turn, tok = n // TOK, n % TOK
                    for h in range(EMB // LANES):
                        stage_v[
                            turn, pl.ds(tok * EMB + h * LANES, LANES)
                        ] = pad_v[half, t, pl.ds(h * LANES, LANES)]
            pltpu.async_copy(stage_v, out_hbm.at[b], sem_w)

            @pl.when(i + 1 < per_w)
            def _():
                pltpu.async_copy(num_hbm.at[b + 1], num_v, sem_num)

            return carry

        lax.fori_loop(0, per_w, body, 0)
        pltpu.make_async_copy(
            stage_v, out_hbm.at[b0 + per_w - 1], sem_w
        ).wait()

    return k


def kernel(token_inputs, numerical_inputs, text_emb_table):
    B, Tn, F = token_inputs.shape
    idx = token_inputs.astype(jnp.int32).reshape(B, NG, GW)
    table_p = jnp.pad(text_emb_table, ((0, 0), (0, 128 - EMB)))
    return _build(B)(idx, numerical_inputs, table_p)
